# baseline (speedup 1.0000x reference)
import jax
import jax.numpy as jnp
from jax.experimental import pallas as pl


def _body(a_ref, b_ref, o_ref):
    a = a_ref[...]
    o_ref[...] = jnp.where(a == 1.0, 0.0, a) + b_ref[...]


def kernel(x_1, x_2):
    orig_shape = x_1.shape
    a = x_1.reshape(-1, 1024)
    b = x_2.reshape(-1, 1024)
    m = a.shape[0]
    bm = 2048
    out = pl.pallas_call(
        _body,
        grid=(m // bm,),
        in_specs=[
            pl.BlockSpec((bm, 1024), lambda i: (i, 0)),
            pl.BlockSpec((bm, 1024), lambda i: (i, 0)),
        ],
        out_specs=pl.BlockSpec((bm, 1024), lambda i: (i, 0)),
        out_shape=jax.ShapeDtypeStruct((m, 1024), jnp.float32),
    )(a, b)
    return out.reshape(orig_shape)
